# baseline (speedup 1.0000x reference)
import jax
import jax.numpy as jnp
from jax.experimental import pallas as pl
from jax.experimental.pallas import tpu as pltpu

_ROWS = 4 * 8192
_LANES = 129 * 2
_BLOCK_ROWS = 2048


def _body(tok_ref, x_ref, o_ref):
    t = tok_ref[0]
    lane = jax.lax.broadcasted_iota(jnp.int32, (_BLOCK_ROWS, _LANES), 1)
    reg = lane // 2
    col0 = (lane % 2) == 0

    is_start = t == 0
    is_digit = (t >= 1) & (t <= 10)
    is_plus = t == 11
    is_minus = t == 12
    is_equals = t == 13
    digit_val = (t - 1) % 10

    x = x_ref[...]
    out = x
    prot = reg < 20
    out = jnp.where(is_start & prot, 0.0, out)
    digit_band = (reg >= 2) & (reg <= 11) & col0
    out = jnp.where(is_digit & digit_band, 0.0, out)
    digit_hit = (reg == 2 + (digit_val % 10)) & col0
    out = jnp.where(is_digit & digit_hit, 1.0, out)
    op_reg = (reg == 1) & col0
    out = jnp.where(is_plus & op_reg, 1.0, out)
    out = jnp.where(is_minus & op_reg, -1.0, out)
    result_regs = (reg >= 14) & (reg <= 16) & col0
    out = jnp.where(is_equals & (result_regs | op_reg | digit_band), 0.0, out)
    o_ref[...] = out


def kernel(carrier_freq, src_token, tgt_token):
    x2d = carrier_freq.reshape(_ROWS, _LANES)
    tok = jnp.asarray(src_token, jnp.int32).reshape(1)
    out = pl.pallas_call(
        _body,
        grid=(_ROWS // _BLOCK_ROWS,),
        in_specs=[
            pl.BlockSpec(memory_space=pltpu.SMEM),
            pl.BlockSpec((_BLOCK_ROWS, _LANES), lambda i: (i, 0)),
        ],
        out_specs=pl.BlockSpec((_BLOCK_ROWS, _LANES), lambda i: (i, 0)),
        out_shape=jax.ShapeDtypeStruct((_ROWS, _LANES), jnp.float32),
    )(tok, x2d)
    return out.reshape(carrier_freq.shape)
